# baseline (speedup 1.0000x reference)
import dataclasses
import functools

import jax
import jax.numpy as jnp
from jax import lax
from jax.experimental import pallas as pl
from jax.experimental.pallas import tpu as pltpu
from jax.experimental.pallas import tpu_sc as plsc

N_NODES = 10000
N_EDGES = 320000
D = 128

NC = 2
NS = 16
CHUNK = 128
NCHUNK = 80
NBUF = 2
IB = 4
EDGES_PER_TILE = CHUNK * NCHUNK
E_PAD = NC * NS * EDGES_PER_TILE
ROWS_PER_TILE = 624
ROWS_TAIL = N_NODES - NS * ROWS_PER_TILE

_FB = D // 16


def _sc_scatter_body(hr_hbm, sd_hbm, wpk_hbm, zeros_hbm, out_hbm,
                     sd_v, wpk_v, st0, st1, rows0, rows1, acc_sh,
                     sem_g0, sem_g1, sem_s0, sem_s1):
    cid = lax.axis_index("c")
    sid = lax.axis_index("s")
    stage = (st0, st1)
    rows = (rows0, rows1)
    sems_g = (sem_g0, sem_g1)
    sems_s = (sem_s0, sem_s1)

    pltpu.sync_copy(zeros_hbm.at[pl.ds(sid * ROWS_PER_TILE, ROWS_PER_TILE)],
                    acc_sh.at[pl.ds(sid * ROWS_PER_TILE, ROWS_PER_TILE)])

    @pl.when(sid == NS - 1)
    def _zero_tail():
        pltpu.sync_copy(zeros_hbm.at[pl.ds(NS * ROWS_PER_TILE, ROWS_TAIL)],
                        acc_sh.at[pl.ds(NS * ROWS_PER_TILE, ROWS_TAIL)])

    pltpu.sync_copy(sd_hbm.at[cid, sid], sd_v)
    pltpu.sync_copy(wpk_hbm.at[cid, sid], wpk_v)

    def unpack_sd(b, j):
        @plsc.parallel_loop(0, _FB)
        def _grp(g):
            sl = pl.ds(g * 16, 16)
            v = sd_v[j, sl]
            stage[b][0, sl] = v & 0x3FFF
            stage[b][1, sl] = v >> 14

    def start_gather(b, j):
        pltpu.async_copy(hr_hbm.at[stage[b].at[0]], rows[b], sems_g[b])

    def wait_gather(b):
        pltpu.make_async_copy(hr_hbm.at[stage[b].at[0]], rows[b],
                              sems_g[b]).wait()

    def start_scatter(b, j):
        pltpu.async_copy(rows[b], acc_sh.at[stage[b].at[1]], sems_s[b],
                         add=True)

    def wait_scatter(b):
        pltpu.make_async_copy(rows[b], acc_sh.at[stage[b].at[1]],
                              sems_s[b]).wait()

    def scale(b, j):
        base_pair = j * (CHUNK // 2)

        @plsc.parallel_loop(0, CHUNK)
        def _edge(e):
            vp = plsc.load_gather(
                wpk_v, [jnp.full((16,), base_pair, jnp.int32) + (e >> 1)])
            lo = plsc.bitcast(vp << 16, jnp.float32)
            hi = plsc.bitcast(vp & jnp.int32(-65536), jnp.float32)
            wv = jnp.where((e & 1) == 1, hi, lo)
            for fb in range(_FB):
                sl = pl.ds(fb * 16, 16)
                rows[b][e, sl] = rows[b][e, sl] * wv

    plsc.subcore_barrier()

    @pl.loop(0, NCHUNK, step=NBUF)
    def _chunks(j0):
        for b in range(NBUF):
            j = j0 + b

            @pl.when(j0 > 0)
            def _drain():
                wait_scatter(b)

            unpack_sd(b, j)
            start_gather(b, j)
            wait_gather(b)
            scale(b, j)
            start_scatter(b, j)

    for b in range(NBUF):
        wait_scatter(b)

    plsc.subcore_barrier()
    pltpu.sync_copy(acc_sh.at[pl.ds(sid * ROWS_PER_TILE, ROWS_PER_TILE)],
                    out_hbm.at[cid, pl.ds(sid * ROWS_PER_TILE, ROWS_PER_TILE)])

    @pl.when(sid == NS - 1)
    def _write_tail():
        pltpu.sync_copy(acc_sh.at[pl.ds(NS * ROWS_PER_TILE, ROWS_TAIL)],
                        out_hbm.at[cid, pl.ds(NS * ROWS_PER_TILE, ROWS_TAIL)])


def _sc_scatter(hr, packed_sd, packed_w, zeros):
    mesh = plsc.VectorSubcoreMesh(core_axis_name="c", subcore_axis_name="s")
    cp = pltpu.CompilerParams()
    if "needs_layout_passes" in pltpu.CompilerParams.__dataclass_fields__:
        cp = dataclasses.replace(cp, needs_layout_passes=False)
    kern = pl.kernel(
        _sc_scatter_body,
        compiler_params=cp,
        out_type=jax.ShapeDtypeStruct((NC, N_NODES, D), jnp.float32),
        mesh=mesh,
        scratch_types=[
            pltpu.VMEM((NCHUNK, CHUNK), jnp.int32),
            pltpu.VMEM((EDGES_PER_TILE // 2,), jnp.int32),
            pltpu.VMEM((2, CHUNK), jnp.int32),
            pltpu.VMEM((2, CHUNK), jnp.int32),
            pltpu.VMEM((CHUNK, D), jnp.float32),
            pltpu.VMEM((CHUNK, D), jnp.float32),
            pltpu.VMEM_SHARED((N_NODES, D), jnp.float32),
            pltpu.SemaphoreType.DMA,
            pltpu.SemaphoreType.DMA,
            pltpu.SemaphoreType.DMA,
            pltpu.SemaphoreType.DMA,
        ],
    )
    return kern(hr, packed_sd, packed_w, zeros)


_BLK = 2000


def _pre_body(h_ref, wr_ref, wo_ref, b_ref, hr_ref, base_ref):
    h = h_ref[...]
    hr_ref[...] = jnp.dot(h, wr_ref[...], preferred_element_type=jnp.float32)
    base_ref[...] = (
        jnp.dot(h, wo_ref[...], preferred_element_type=jnp.float32)
        + b_ref[...]
    )


def _dense_pre(h, w_rel, w_root, b_rel):
    return pl.pallas_call(
        _pre_body,
        grid=(N_NODES // _BLK,),
        in_specs=[
            pl.BlockSpec((_BLK, D), lambda i: (i, 0)),
            pl.BlockSpec((D, D), lambda i: (0, 0)),
            pl.BlockSpec((D, D), lambda i: (0, 0)),
            pl.BlockSpec((1, D), lambda i: (0, 0)),
        ],
        out_specs=[
            pl.BlockSpec((_BLK, D), lambda i: (i, 0)),
            pl.BlockSpec((_BLK, D), lambda i: (i, 0)),
        ],
        out_shape=[
            jax.ShapeDtypeStruct((N_NODES, D), jnp.float32),
            jax.ShapeDtypeStruct((N_NODES, D), jnp.float32),
        ],
    )(h, w_rel, w_root, b_rel.reshape(1, D))


def _mid_body(p_ref, base_ref, wr_ref, wo_ref, b_ref, hr_ref, base2_ref):
    h = p_ref[0] + p_ref[1] + base_ref[...]
    h = jnp.where(h > 0, h, jnp.exp(jnp.minimum(h, 0.0)) - 1.0)
    hr_ref[...] = jnp.dot(h, wr_ref[...], preferred_element_type=jnp.float32)
    base2_ref[...] = (
        jnp.dot(h, wo_ref[...], preferred_element_type=jnp.float32)
        + b_ref[...]
    )


def _dense_mid(p, base, w_rel, w_root, b_rel):
    return pl.pallas_call(
        _mid_body,
        grid=(N_NODES // _BLK,),
        in_specs=[
            pl.BlockSpec((NC, _BLK, D), lambda i: (0, i, 0)),
            pl.BlockSpec((_BLK, D), lambda i: (i, 0)),
            pl.BlockSpec((D, D), lambda i: (0, 0)),
            pl.BlockSpec((D, D), lambda i: (0, 0)),
            pl.BlockSpec((1, D), lambda i: (0, 0)),
        ],
        out_specs=[
            pl.BlockSpec((_BLK, D), lambda i: (i, 0)),
            pl.BlockSpec((_BLK, D), lambda i: (i, 0)),
        ],
        out_shape=[
            jax.ShapeDtypeStruct((N_NODES, D), jnp.float32),
            jax.ShapeDtypeStruct((N_NODES, D), jnp.float32),
        ],
    )(p, base, w_rel, w_root, b_rel.reshape(1, D))


def _final_body(p_ref, base_ref, out_ref):
    out_ref[...] = p_ref[0] + p_ref[1] + base_ref[...]


def _dense_final(p, base):
    return pl.pallas_call(
        _final_body,
        grid=(N_NODES // _BLK,),
        in_specs=[
            pl.BlockSpec((NC, _BLK, D), lambda i: (0, i, 0)),
            pl.BlockSpec((_BLK, D), lambda i: (i, 0)),
        ],
        out_specs=pl.BlockSpec((_BLK, D), lambda i: (i, 0)),
        out_shape=jax.ShapeDtypeStruct((N_NODES, D), jnp.float32),
    )(p, base)


def kernel(x, edge_index, edge_weights, W1_rel, b1_rel, W1_root,
           W2_rel, b2_rel, W2_root):
    ei = edge_index.astype(jnp.int32)
    pad = E_PAD - N_EDGES
    src_p = jnp.pad(ei[0], (0, pad))
    dst_p = jnp.pad(ei[1], (0, pad))
    packed_sd = (src_p | (dst_p << 14)).reshape(NC, NS, NCHUNK, CHUNK)
    w_bf = jnp.pad(edge_weights.astype(jnp.float32), (0, pad)).astype(
        jnp.bfloat16)
    packed_w = lax.bitcast_convert_type(
        w_bf.reshape(NC, NS, EDGES_PER_TILE // 2, 2), jnp.int32)
    zeros = jnp.zeros((N_NODES, D), jnp.float32)

    hr1, base1 = _dense_pre(x, W1_rel, W1_root, b1_rel)
    part1 = _sc_scatter(hr1, packed_sd, packed_w, zeros)
    hr2, base2 = _dense_mid(part1, base1, W2_rel, W2_root, b2_rel)
    part2 = _sc_scatter(hr2, packed_sd, packed_w, zeros)
    return _dense_final(part2, base2)


# speedup vs baseline: 1.2567x; 1.2567x over previous
import dataclasses
import functools

import jax
import jax.numpy as jnp
from jax import lax
from jax.experimental import pallas as pl
from jax.experimental.pallas import tpu as pltpu
from jax.experimental.pallas import tpu_sc as plsc

N_NODES = 10000
N_EDGES = 320000
D = 128

NC = 2
NS = 16
CHUNK = 128
NCH = (79, 79)
NCHMAX = max(NCH)
EDGES_C = tuple(n * CHUNK for n in NCH)
ROWS_PER_TILE = 624
ROWS_TAIL = N_NODES - NS * ROWS_PER_TILE

_FB = D // 16


def _sc_scatter_body(hr_hbm, src_hbm, dst_hbm, wpk_hbm, zeros_hbm, out_hbm,
                     src_v, dst_v, wpk_v, rows_v, acc_sh, sem):
    cid = lax.axis_index("c")
    sid = lax.axis_index("s")

    pltpu.sync_copy(zeros_hbm.at[pl.ds(sid * ROWS_PER_TILE, ROWS_PER_TILE)],
                    acc_sh.at[pl.ds(sid * ROWS_PER_TILE, ROWS_PER_TILE)])

    @pl.when(sid == NS - 1)
    def _zero_tail():
        pltpu.sync_copy(zeros_hbm.at[pl.ds(NS * ROWS_PER_TILE, ROWS_TAIL)],
                        acc_sh.at[pl.ds(NS * ROWS_PER_TILE, ROWS_TAIL)])

    plsc.subcore_barrier()

    pltpu.sync_copy(src_hbm.at[cid, sid], src_v)
    pltpu.sync_copy(dst_hbm.at[cid, sid], dst_v)
    pltpu.sync_copy(wpk_hbm.at[cid, sid], wpk_v)

    nchunk = jnp.where(cid == 0, NCH[0], NCH[1])

    @pl.loop(0, nchunk)
    def _chunk(j):
        pltpu.async_copy(hr_hbm.at[src_v.at[j]], rows_v, sem).wait()
        base_pair = j * (CHUNK // 2)

        @pl.loop(0, CHUNK)
        def _edge(e):
            vp = plsc.load_gather(
                wpk_v, [jnp.full((16,), base_pair, jnp.int32) + (e >> 1)])
            lo = plsc.bitcast(vp << 16, jnp.float32)
            hi = plsc.bitcast(vp & jnp.int32(-65536), jnp.float32)
            wv = jnp.where((e & 1) == 1, hi, lo)
            for fb in range(_FB):
                sl = pl.ds(fb * 16, 16)
                rows_v[e, sl] = rows_v[e, sl] * wv

        pltpu.sync_copy(rows_v, acc_sh.at[dst_v.at[j]], add=True)

    plsc.subcore_barrier()
    pltpu.sync_copy(acc_sh.at[pl.ds(sid * ROWS_PER_TILE, ROWS_PER_TILE)],
                    out_hbm.at[cid, pl.ds(sid * ROWS_PER_TILE, ROWS_PER_TILE)])

    @pl.when(sid == NS - 1)
    def _write_tail():
        pltpu.sync_copy(acc_sh.at[pl.ds(NS * ROWS_PER_TILE, ROWS_TAIL)],
                        out_hbm.at[cid, pl.ds(NS * ROWS_PER_TILE, ROWS_TAIL)])


def _sc_scatter(hr, src_p, dst_p, w_pk, zeros):
    mesh = plsc.VectorSubcoreMesh(core_axis_name="c", subcore_axis_name="s")
    cp = pltpu.CompilerParams()
    if "needs_layout_passes" in pltpu.CompilerParams.__dataclass_fields__:
        cp = dataclasses.replace(cp, needs_layout_passes=False)
    kern = pl.kernel(
        _sc_scatter_body,
        compiler_params=cp,
        out_type=jax.ShapeDtypeStruct((NC, N_NODES, D), jnp.float32),
        mesh=mesh,
        scratch_types=[
            pltpu.VMEM((NCHMAX, CHUNK), jnp.int32),
            pltpu.VMEM((NCHMAX, CHUNK), jnp.int32),
            pltpu.VMEM((NCHMAX * CHUNK // 2,), jnp.int32),
            pltpu.VMEM((CHUNK, D), jnp.float32),
            pltpu.VMEM_SHARED((N_NODES, D), jnp.float32),
            pltpu.SemaphoreType.DMA,
        ],
    )
    return kern(hr, src_p, dst_p, w_pk, zeros)


_BLK = 2000


def _pre_body(h_ref, wr_ref, wo_ref, b_ref, hr_ref, base_ref):
    h = h_ref[...]
    hr_ref[...] = jnp.dot(h, wr_ref[...], preferred_element_type=jnp.float32)
    base_ref[...] = (
        jnp.dot(h, wo_ref[...], preferred_element_type=jnp.float32)
        + b_ref[...]
    )


def _dense_pre(h, w_rel, w_root, b_rel):
    return pl.pallas_call(
        _pre_body,
        grid=(N_NODES // _BLK,),
        in_specs=[
            pl.BlockSpec((_BLK, D), lambda i: (i, 0)),
            pl.BlockSpec((D, D), lambda i: (0, 0)),
            pl.BlockSpec((D, D), lambda i: (0, 0)),
            pl.BlockSpec((1, D), lambda i: (0, 0)),
        ],
        out_specs=[
            pl.BlockSpec((_BLK, D), lambda i: (i, 0)),
            pl.BlockSpec((_BLK, D), lambda i: (i, 0)),
        ],
        out_shape=[
            jax.ShapeDtypeStruct((N_NODES, D), jnp.float32),
            jax.ShapeDtypeStruct((N_NODES, D), jnp.float32),
        ],
    )(h, w_rel, w_root, b_rel.reshape(1, D))


def _mid_body(p_ref, base_ref, wr_ref, wo_ref, b_ref, hr_ref, base2_ref):
    h = p_ref[0] + p_ref[1] + base_ref[...]
    h = jnp.where(h > 0, h, jnp.exp(jnp.minimum(h, 0.0)) - 1.0)
    hr_ref[...] = jnp.dot(h, wr_ref[...], preferred_element_type=jnp.float32)
    base2_ref[...] = (
        jnp.dot(h, wo_ref[...], preferred_element_type=jnp.float32)
        + b_ref[...]
    )


def _dense_mid(p, base, w_rel, w_root, b_rel):
    return pl.pallas_call(
        _mid_body,
        grid=(N_NODES // _BLK,),
        in_specs=[
            pl.BlockSpec((NC, _BLK, D), lambda i: (0, i, 0)),
            pl.BlockSpec((_BLK, D), lambda i: (i, 0)),
            pl.BlockSpec((D, D), lambda i: (0, 0)),
            pl.BlockSpec((D, D), lambda i: (0, 0)),
            pl.BlockSpec((1, D), lambda i: (0, 0)),
        ],
        out_specs=[
            pl.BlockSpec((_BLK, D), lambda i: (i, 0)),
            pl.BlockSpec((_BLK, D), lambda i: (i, 0)),
        ],
        out_shape=[
            jax.ShapeDtypeStruct((N_NODES, D), jnp.float32),
            jax.ShapeDtypeStruct((N_NODES, D), jnp.float32),
        ],
    )(p, base, w_rel, w_root, b_rel.reshape(1, D))


def _final_body(p_ref, base_ref, out_ref):
    out_ref[...] = p_ref[0] + p_ref[1] + base_ref[...]


def _dense_final(p, base):
    return pl.pallas_call(
        _final_body,
        grid=(N_NODES // _BLK,),
        in_specs=[
            pl.BlockSpec((NC, _BLK, D), lambda i: (0, i, 0)),
            pl.BlockSpec((_BLK, D), lambda i: (i, 0)),
        ],
        out_specs=pl.BlockSpec((_BLK, D), lambda i: (i, 0)),
        out_shape=jax.ShapeDtypeStruct((N_NODES, D), jnp.float32),
    )(p, base)


def _split_pad(a, fill):
    n0 = NS * EDGES_C[0]
    n1 = NS * EDGES_C[1]
    a = jnp.pad(a, (0, n0 + n1 - N_EDGES), constant_values=fill)
    parts = []
    for c, (n, epc) in enumerate(((n0, EDGES_C[0]), (n1, EDGES_C[1]))):
        s = a[n0 * c: n0 * c + n].reshape(NS, epc)
        s = jnp.pad(s, ((0, 0), (0, NCHMAX * CHUNK - epc)),
                    constant_values=fill)
        parts.append(s)
    return jnp.stack(parts, axis=0)


def kernel(x, edge_index, edge_weights, W1_rel, b1_rel, W1_root,
           W2_rel, b2_rel, W2_root):
    ei = edge_index.astype(jnp.int32)
    src_p = _split_pad(ei[0], 0).reshape(NC, NS, NCHMAX, CHUNK)
    dst_p = _split_pad(ei[1], 0).reshape(NC, NS, NCHMAX, CHUNK)
    w_bf = _split_pad(edge_weights.astype(jnp.float32), 0.0).astype(
        jnp.bfloat16)
    w_pk = lax.bitcast_convert_type(
        w_bf.reshape(NC, NS, NCHMAX * CHUNK // 2, 2), jnp.int32)
    zeros = jnp.zeros((N_NODES, D), jnp.float32)

    hr1, base1 = _dense_pre(x, W1_rel, W1_root, b1_rel)
    part1 = _sc_scatter(hr1, src_p, dst_p, w_pk, zeros)
    hr2, base2 = _dense_mid(part1, base1, W2_rel, W2_root, b2_rel)
    part2 = _sc_scatter(hr2, src_p, dst_p, w_pk, zeros)
    return _dense_final(part2, base2)
